# baseline (speedup 1.0000x reference)
import jax
import jax.numpy as jnp
from jax.experimental import pallas as pl

_NUM_FIELDS = 26
_DEPTH = 1000
_BD = 200
_SPF = _DEPTH // _BD


def _onehot_t_block(fvt_ref, out_ref):
    i = pl.program_id(0)
    fv_row = fvt_ref[0]
    base = jax.lax.rem(i, _SPF) * _BD
    pos = base + jax.lax.broadcasted_iota(jnp.int32, out_ref.shape, 0)
    out_ref[...] = (pos == fv_row).astype(jnp.float32)


def kernel(feature_value):
    batch = feature_value.shape[0]
    fvt = feature_value.T.reshape(_NUM_FIELDS, 1, batch)
    out_t = pl.pallas_call(
        _onehot_t_block,
        grid=(_NUM_FIELDS * _SPF,),
        in_specs=[pl.BlockSpec((1, 1, batch), lambda i: (i // _SPF, 0, 0))],
        out_specs=pl.BlockSpec((_BD, batch), lambda i: (i, 0)),
        out_shape=jax.ShapeDtypeStruct((_NUM_FIELDS * _DEPTH, batch),
                                       jnp.float32),
    )(fvt)
    return out_t.T


# speedup vs baseline: 1.0228x; 1.0228x over previous
import jax
import jax.numpy as jnp
from jax.experimental import pallas as pl

_NUM_FIELDS = 26
_DEPTH = 1000


def _onehot_t_block(fvt_ref, out_ref):
    fv_row = fvt_ref[0]
    pos = jax.lax.broadcasted_iota(jnp.int32, out_ref.shape, 0)
    out_ref[...] = (pos == fv_row).astype(jnp.float32)


def kernel(feature_value):
    batch = feature_value.shape[0]
    fvt = feature_value.T.reshape(_NUM_FIELDS, 1, batch)
    out_t = pl.pallas_call(
        _onehot_t_block,
        grid=(_NUM_FIELDS,),
        in_specs=[pl.BlockSpec((1, 1, batch), lambda f: (f, 0, 0))],
        out_specs=pl.BlockSpec((_DEPTH, batch), lambda f: (f, 0)),
        out_shape=jax.ShapeDtypeStruct((_NUM_FIELDS * _DEPTH, batch),
                                       jnp.float32),
    )(fvt)
    return out_t.T
